# Initial kernel scaffold; baseline (speedup 1.0000x reference)
import jax
import jax.numpy as jnp
from jax.experimental import pallas as pl


def kernel(tile_inputs, poi_inputs, road_edges, tree_edges, contains_edges, params):
    raise NotImplementedError("write your pallas kernel here")


# speedup vs baseline: 23.7870x; 23.7870x over previous
import functools

import numpy as np

import jax
import jax.numpy as jnp
from jax import lax
from jax.experimental import pallas as pl
from jax.experimental.pallas import tpu as pltpu
from jax.experimental.pallas import tpu_sc as plsc

N_NODES = 10000
FEATS = 128
HEADS = 8
LANES = 16
E_TOTAL = 640000
NC = 2
NS = 16
NW = NC * NS
E_PER_W = E_TOTAL // NW
CHUNK = 80
N_CHUNKS = E_PER_W // CHUNK
N_PAD = 10240
ROWS_PER_SUB = N_PAD // NS
_ZFULL = ROWS_PER_SUB // CHUNK

_E_SEL = np.zeros((16, FEATS), np.float32)
for _h in range(HEADS):
    _E_SEL[_h, _h * LANES:(_h + 1) * LANES] = 1.0

_SC_MESH = plsc.VectorSubcoreMesh(core_axis_name="c", subcore_axis_name="s")


@functools.partial(
    pl.kernel,
    mesh=_SC_MESH,
    out_type=[
        jax.ShapeDtypeStruct((NC, N_PAD, FEATS), jnp.float32),
        jax.ShapeDtypeStruct((NC, N_PAD, LANES), jnp.float32),
    ],
    scratch_types=[
        pltpu.VMEM((CHUNK,), jnp.int32),
        pltpu.VMEM((CHUNK,), jnp.int32),
        pltpu.VMEM((CHUNK, FEATS), jnp.float32),
        pltpu.VMEM((CHUNK, FEATS), jnp.float32),
        pltpu.VMEM((CHUNK, LANES), jnp.float32),
        pltpu.VMEM((HEADS, LANES), jnp.float32),
        pltpu.VMEM_SHARED((N_PAD, FEATS), jnp.float32),
        pltpu.VMEM_SHARED((N_PAD, LANES), jnp.float32),
        pltpu.SemaphoreType.DMA,
        pltpu.SemaphoreType.DMA,
    ],
)
def _edge_pass(fs_hbm, fd_hbm, src_hbm, dst_hbm, attn_hbm,
               num_out, den_out,
               src_v, dst_v, fs_rows, fd_rows, exden, attn_v,
               num_sh, den_sh, sem_fs, sem_fd):
    cid = lax.axis_index("c")
    sid = lax.axis_index("s")
    wid = sid * NC + cid

    zv = jnp.zeros((LANES,), jnp.float32)

    def zero_body(e, carry):
        for q in range(FEATS // LANES):
            fs_rows[e, pl.ds(q * LANES, LANES)] = zv
        exden[e, :] = zv
        return carry

    lax.fori_loop(0, CHUNK, zero_body, 0)

    row0 = sid * ROWS_PER_SUB

    lane16 = lax.iota(jnp.int32, LANES)
    for k in range(_ZFULL):
        base = row0 + k * CHUNK
        for q in range(CHUNK // LANES):
            src_v[pl.ds(q * LANES, LANES)] = base + q * LANES + lane16
        pltpu.sync_copy(fs_rows, num_sh.at[src_v])
        pltpu.sync_copy(exden, den_sh.at[src_v])
    pltpu.sync_copy(attn_hbm, attn_v)
    plsc.subcore_barrier()

    attn = [attn_v[h, :] for h in range(HEADS)]
    lane = lax.iota(jnp.int32, LANES)
    ohs = [jnp.where(lane == h, 1.0, 0.0).astype(jnp.float32)
           for h in range(HEADS)]
    xperm = [jnp.bitwise_xor(lane, k) for k in (8, 4, 2, 1)]

    def lane_sum(p):
        for perm in xperm:
            p = p + p.at[perm].get(mode="promise_in_bounds")
        return p

    def chunk_body(i, carry):
        base = wid * E_PER_W + i * CHUNK
        pltpu.sync_copy(src_hbm.at[pl.ds(base, CHUNK)], src_v)
        pltpu.sync_copy(dst_hbm.at[pl.ds(base, CHUNK)], dst_v)
        pltpu.async_copy(fs_hbm.at[src_v], fs_rows, sem_fs).wait()
        pltpu.async_copy(fd_hbm.at[dst_v], fd_rows, sem_fd).wait()

        def edge_body(e, ecarry):
            acc = jnp.zeros((LANES,), jnp.float32)
            for h in range(HEADS):
                u = fs_rows[e, pl.ds(h * LANES, LANES)]
                v = fd_rows[e, pl.ds(h * LANES, LANES)]
                s = u + v
                t = jnp.where(s >= 0.0, s, s * 0.2)
                ex = jnp.exp(lane_sum(t * attn[h]))
                fs_rows[e, pl.ds(h * LANES, LANES)] = u * ex
                acc = acc + ex * ohs[h]
            exden[e, :] = acc
            return ecarry

        lax.fori_loop(0, CHUNK, edge_body, 0)

        pltpu.sync_copy(fs_rows, num_sh.at[dst_v], add=True)
        pltpu.sync_copy(exden, den_sh.at[dst_v], add=True)
        return carry

    lax.fori_loop(0, N_CHUNKS, chunk_body, 0)

    plsc.subcore_barrier()
    for k in range(_ZFULL):
        base = row0 + k * CHUNK
        for q in range(CHUNK // LANES):
            src_v[pl.ds(q * LANES, LANES)] = base + q * LANES + lane16
        pltpu.async_copy(num_sh.at[src_v], fs_rows, sem_fs).wait()
        pltpu.async_copy(den_sh.at[src_v], exden, sem_fd).wait()
        pltpu.sync_copy(fs_rows, num_out.at[cid, pl.ds(base, CHUNK)])
        pltpu.sync_copy(exden, den_out.at[cid, pl.ds(base, CHUNK)])


_BM = 2000


def _proj_body(x_ref, w_ref, b_ref, o1_ref, o2_ref, o3_ref):
    r = jnp.dot(x_ref[...], w_ref[...],
                preferred_element_type=jnp.float32) + b_ref[...]
    o1_ref[...] = r[:, :FEATS]
    o2_ref[...] = r[:, FEATS:2 * FEATS]
    o3_ref[...] = r[:, 2 * FEATS:]


def _proj(x, w, b):
    m = x.shape[0]
    n = w.shape[1]
    outs = pl.pallas_call(
        _proj_body,
        grid=(m // _BM,),
        in_specs=[
            pl.BlockSpec((_BM, FEATS), lambda i: (i, 0)),
            pl.BlockSpec((FEATS, n), lambda i: (0, 0)),
            pl.BlockSpec((1, n), lambda i: (0, 0)),
        ],
        out_specs=[pl.BlockSpec((_BM, FEATS), lambda i: (i, 0))] * 3,
        out_shape=[jax.ShapeDtypeStruct((m, FEATS), jnp.float32)] * 3,
    )(x, w, b.reshape(1, n))
    return outs


def _fin1_body(num_ref, den_ref, e_ref, o_ref):
    a = num_ref[...]
    d = den_ref[...]
    n = a[0] + a[1]
    dx = jnp.dot(d[0] + d[1], e_ref[...], preferred_element_type=jnp.float32)
    o_ref[...] = jnp.maximum(n / jnp.maximum(dx, 1e-16), 0.0)


def _fin1(num, den, e_sel):
    return pl.pallas_call(
        _fin1_body,
        grid=(N_NODES // _BM,),
        in_specs=[
            pl.BlockSpec((NC, _BM, FEATS), lambda i: (0, i, 0)),
            pl.BlockSpec((NC, _BM, LANES), lambda i: (0, i, 0)),
            pl.BlockSpec((LANES, FEATS), lambda i: (0, 0)),
        ],
        out_specs=pl.BlockSpec((_BM, FEATS), lambda i: (i, 0)),
        out_shape=jax.ShapeDtypeStruct((N_NODES, FEATS), jnp.float32),
    )(num, den, e_sel)


def _fin2_body(n1_ref, d1_ref, n2_ref, d2_ref, e_ref, o_ref, *, scale):
    e = e_ref[...]
    a1 = n1_ref[...]
    b1 = d1_ref[...]
    a2 = n2_ref[...]
    b2 = d2_ref[...]
    x1 = (a1[0] + a1[1]) / jnp.maximum(
        jnp.dot(b1[0] + b1[1], e, preferred_element_type=jnp.float32), 1e-16)
    x2 = (a2[0] + a2[1]) / jnp.maximum(
        jnp.dot(b2[0] + b2[1], e, preferred_element_type=jnp.float32), 1e-16)
    o_ref[...] = jnp.maximum((x1 + x2) * scale, 0.0)


def _fin2(num1, den1, num2, den2, e_sel, scale):
    return pl.pallas_call(
        functools.partial(_fin2_body, scale=scale),
        grid=(N_NODES // _BM,),
        in_specs=[
            pl.BlockSpec((NC, _BM, FEATS), lambda i: (0, i, 0)),
            pl.BlockSpec((NC, _BM, LANES), lambda i: (0, i, 0)),
            pl.BlockSpec((NC, _BM, FEATS), lambda i: (0, i, 0)),
            pl.BlockSpec((NC, _BM, LANES), lambda i: (0, i, 0)),
            pl.BlockSpec((LANES, FEATS), lambda i: (0, 0)),
        ],
        out_specs=pl.BlockSpec((_BM, FEATS), lambda i: (i, 0)),
        out_shape=jax.ShapeDtypeStruct((N_NODES, FEATS), jnp.float32),
    )(num1, den1, num2, den2, e_sel)


def kernel(tile_inputs, poi_inputs, road_edges, tree_edges, contains_edges,
           params):
    e_sel = jnp.asarray(_E_SEL)
    road_src, road_dst = road_edges[0], road_edges[1]
    tree_src, tree_dst = tree_edges[0], tree_edges[1]
    cont_src, cont_dst = contains_edges[0], contains_edges[1]

    h_t, h_p = tile_inputs, poi_inputs
    for li, layer in enumerate(params['layers']):
        pr, pt, pc = layer['road'], layer['tree_branch'], layer['contains']
        w_t = jnp.concatenate([pr['W_src'], pr['W_dst'], pc['W_src']], axis=1)
        b_t = jnp.concatenate([pr['b_src'], pr['b_dst'], pc['b_src']])
        w_p = jnp.concatenate([pt['W_src'], pt['W_dst'], pc['W_dst']], axis=1)
        b_p = jnp.concatenate([pt['b_src'], pt['b_dst'], pc['b_dst']])

        fs_road, fd_road, fs_cont = _proj(h_t, w_t, b_t)
        fs_tree, fd_tree, fd_cont = _proj(h_p, w_p, b_p)

        num_r, den_r = _edge_pass(fs_road, fd_road, road_src, road_dst,
                                  pr['attn'])
        num_t, den_t = _edge_pass(fs_tree, fd_tree, tree_src, tree_dst,
                                  pt['attn'])
        num_c, den_c = _edge_pass(fs_cont, fd_cont, cont_src, cont_dst,
                                  pc['attn'])

        h_t = _fin1(num_r, den_r, e_sel)
        scale = 0.5 if li == 1 else 1.0
        h_p = _fin2(num_t, den_t, num_c, den_c, e_sel, scale)
    return h_t, h_p


# speedup vs baseline: 27.2105x; 1.1439x over previous
import functools

import numpy as np

import jax
import jax.numpy as jnp
from jax import lax
from jax.experimental import pallas as pl
from jax.experimental.pallas import tpu as pltpu
from jax.experimental.pallas import tpu_sc as plsc

N_NODES = 10000
FEATS = 128
HEADS = 8
LANES = 16
E_TOTAL = 640000
NC = 2
NS = 16
NW = NC * NS
E_PER_W = E_TOTAL // NW
CHUNK = 80
N_CHUNKS = E_PER_W // CHUNK
N_PAD = 10240
ROWS_PER_SUB = N_PAD // NS
_ZFULL = ROWS_PER_SUB // CHUNK

_E_SEL = np.zeros((16, FEATS), np.float32)
for _h in range(HEADS):
    _E_SEL[_h, _h * LANES:(_h + 1) * LANES] = 1.0

_SC_MESH = plsc.VectorSubcoreMesh(core_axis_name="c", subcore_axis_name="s")


@functools.partial(
    pl.kernel,
    mesh=_SC_MESH,
    out_type=[
        jax.ShapeDtypeStruct((NC, N_PAD, FEATS), jnp.float32),
        jax.ShapeDtypeStruct((NC, N_PAD, LANES), jnp.float32),
    ],
    scratch_types=[
        pltpu.VMEM((CHUNK,), jnp.int32),
        pltpu.VMEM((CHUNK,), jnp.int32),
        pltpu.VMEM((CHUNK, FEATS), jnp.float32),
        pltpu.VMEM((CHUNK, FEATS), jnp.float32),
        pltpu.VMEM((CHUNK, LANES), jnp.float32),
        pltpu.VMEM((HEADS, LANES), jnp.float32),
        pltpu.VMEM_SHARED((N_PAD, FEATS), jnp.float32),
        pltpu.VMEM_SHARED((N_PAD, LANES), jnp.float32),
        pltpu.SemaphoreType.DMA,
        pltpu.SemaphoreType.DMA,
    ],
)
def _edge_pass(fs_hbm, fd_hbm, src_hbm, dst_hbm, attn_hbm,
               num_out, den_out,
               src_v, dst_v, fs_rows, fd_rows, exden, attn_v,
               num_sh, den_sh, sem_fs, sem_fd):
    cid = lax.axis_index("c")
    sid = lax.axis_index("s")
    wid = sid * NC + cid

    zv = jnp.zeros((LANES,), jnp.float32)

    def zero_body(e, carry):
        for q in range(FEATS // LANES):
            fs_rows[e, pl.ds(q * LANES, LANES)] = zv
        exden[e, :] = zv
        return carry

    lax.fori_loop(0, CHUNK, zero_body, 0)

    row0 = sid * ROWS_PER_SUB

    lane16 = lax.iota(jnp.int32, LANES)
    for k in range(_ZFULL):
        base = row0 + k * CHUNK
        for q in range(CHUNK // LANES):
            src_v[pl.ds(q * LANES, LANES)] = base + q * LANES + lane16
        pltpu.sync_copy(fs_rows, num_sh.at[src_v])
        pltpu.sync_copy(exden, den_sh.at[src_v])
    pltpu.sync_copy(attn_hbm, attn_v)
    plsc.subcore_barrier()

    attn = [attn_v[h, :] for h in range(HEADS)]
    lane = lax.iota(jnp.int32, LANES)
    ohs = [jnp.where(lane == h, 1.0, 0.0).astype(jnp.float32)
           for h in range(HEADS)]
    xperm = [jnp.bitwise_xor(lane, k) for k in (8, 4, 2, 1)]

    def lane_sum(p):
        for perm in xperm:
            p = p + p.at[perm].get(mode="promise_in_bounds")
        return p

    def chunk_body(i, carry):
        base = wid * E_PER_W + i * CHUNK
        pltpu.sync_copy(src_hbm.at[pl.ds(base, CHUNK)], src_v)
        pltpu.sync_copy(dst_hbm.at[pl.ds(base, CHUNK)], dst_v)
        gfs = pltpu.async_copy(fs_hbm.at[src_v], fs_rows, sem_fs)
        gfd = pltpu.async_copy(fd_hbm.at[dst_v], fd_rows, sem_fd)
        gfs.wait()
        gfd.wait()

        def edge_body(e, ecarry):
            acc = jnp.zeros((LANES,), jnp.float32)
            for h in range(HEADS):
                u = fs_rows[e, pl.ds(h * LANES, LANES)]
                v = fd_rows[e, pl.ds(h * LANES, LANES)]
                s = u + v
                t = jnp.where(s >= 0.0, s, s * 0.2)
                ex = jnp.exp(lane_sum(t * attn[h]))
                fs_rows[e, pl.ds(h * LANES, LANES)] = u * ex
                acc = acc + ex * ohs[h]
            exden[e, :] = acc
            return ecarry

        lax.fori_loop(0, CHUNK, edge_body, 0)

        pltpu.sync_copy(fs_rows, num_sh.at[dst_v], add=True)
        pltpu.sync_copy(exden, den_sh.at[dst_v], add=True)
        return carry

    lax.fori_loop(0, N_CHUNKS, chunk_body, 0)

    plsc.subcore_barrier()
    for k in range(_ZFULL):
        base = row0 + k * CHUNK
        for q in range(CHUNK // LANES):
            src_v[pl.ds(q * LANES, LANES)] = base + q * LANES + lane16
        pltpu.async_copy(num_sh.at[src_v], fs_rows, sem_fs).wait()
        pltpu.async_copy(den_sh.at[src_v], exden, sem_fd).wait()
        pltpu.sync_copy(fs_rows, num_out.at[cid, pl.ds(base, CHUNK)])
        pltpu.sync_copy(exden, den_out.at[cid, pl.ds(base, CHUNK)])


_BM = 2000


def _proj_body(x_ref, w_ref, b_ref, o1_ref, o2_ref, o3_ref):
    r = jnp.dot(x_ref[...], w_ref[...],
                preferred_element_type=jnp.float32) + b_ref[...]
    o1_ref[...] = r[:, :FEATS]
    o2_ref[...] = r[:, FEATS:2 * FEATS]
    o3_ref[...] = r[:, 2 * FEATS:]


def _proj(x, w, b):
    m = x.shape[0]
    n = w.shape[1]
    outs = pl.pallas_call(
        _proj_body,
        grid=(m // _BM,),
        in_specs=[
            pl.BlockSpec((_BM, FEATS), lambda i: (i, 0)),
            pl.BlockSpec((FEATS, n), lambda i: (0, 0)),
            pl.BlockSpec((1, n), lambda i: (0, 0)),
        ],
        out_specs=[pl.BlockSpec((_BM, FEATS), lambda i: (i, 0))] * 3,
        out_shape=[jax.ShapeDtypeStruct((m, FEATS), jnp.float32)] * 3,
    )(x, w, b.reshape(1, n))
    return outs


def _fin1_body(num_ref, den_ref, e_ref, o_ref):
    a = num_ref[...]
    d = den_ref[...]
    n = a[0] + a[1]
    dx = jnp.dot(d[0] + d[1], e_ref[...], preferred_element_type=jnp.float32)
    o_ref[...] = jnp.maximum(n / jnp.maximum(dx, 1e-16), 0.0)


def _fin1(num, den, e_sel):
    return pl.pallas_call(
        _fin1_body,
        grid=(N_NODES // _BM,),
        in_specs=[
            pl.BlockSpec((NC, _BM, FEATS), lambda i: (0, i, 0)),
            pl.BlockSpec((NC, _BM, LANES), lambda i: (0, i, 0)),
            pl.BlockSpec((LANES, FEATS), lambda i: (0, 0)),
        ],
        out_specs=pl.BlockSpec((_BM, FEATS), lambda i: (i, 0)),
        out_shape=jax.ShapeDtypeStruct((N_NODES, FEATS), jnp.float32),
    )(num, den, e_sel)


def _fin2_body(n1_ref, d1_ref, n2_ref, d2_ref, e_ref, o_ref, *, scale):
    e = e_ref[...]
    a1 = n1_ref[...]
    b1 = d1_ref[...]
    a2 = n2_ref[...]
    b2 = d2_ref[...]
    x1 = (a1[0] + a1[1]) / jnp.maximum(
        jnp.dot(b1[0] + b1[1], e, preferred_element_type=jnp.float32), 1e-16)
    x2 = (a2[0] + a2[1]) / jnp.maximum(
        jnp.dot(b2[0] + b2[1], e, preferred_element_type=jnp.float32), 1e-16)
    o_ref[...] = jnp.maximum((x1 + x2) * scale, 0.0)


def _fin2(num1, den1, num2, den2, e_sel, scale):
    return pl.pallas_call(
        functools.partial(_fin2_body, scale=scale),
        grid=(N_NODES // _BM,),
        in_specs=[
            pl.BlockSpec((NC, _BM, FEATS), lambda i: (0, i, 0)),
            pl.BlockSpec((NC, _BM, LANES), lambda i: (0, i, 0)),
            pl.BlockSpec((NC, _BM, FEATS), lambda i: (0, i, 0)),
            pl.BlockSpec((NC, _BM, LANES), lambda i: (0, i, 0)),
            pl.BlockSpec((LANES, FEATS), lambda i: (0, 0)),
        ],
        out_specs=pl.BlockSpec((_BM, FEATS), lambda i: (i, 0)),
        out_shape=jax.ShapeDtypeStruct((N_NODES, FEATS), jnp.float32),
    )(num1, den1, num2, den2, e_sel)


def kernel(tile_inputs, poi_inputs, road_edges, tree_edges, contains_edges,
           params):
    e_sel = jnp.asarray(_E_SEL)
    road_src, road_dst = road_edges[0], road_edges[1]
    tree_src, tree_dst = tree_edges[0], tree_edges[1]
    cont_src, cont_dst = contains_edges[0], contains_edges[1]

    h_t, h_p = tile_inputs, poi_inputs
    for li, layer in enumerate(params['layers']):
        pr, pt, pc = layer['road'], layer['tree_branch'], layer['contains']
        w_t = jnp.concatenate([pr['W_src'], pr['W_dst'], pc['W_src']], axis=1)
        b_t = jnp.concatenate([pr['b_src'], pr['b_dst'], pc['b_src']])
        w_p = jnp.concatenate([pt['W_src'], pt['W_dst'], pc['W_dst']], axis=1)
        b_p = jnp.concatenate([pt['b_src'], pt['b_dst'], pc['b_dst']])

        fs_road, fd_road, fs_cont = _proj(h_t, w_t, b_t)
        fs_tree, fd_tree, fd_cont = _proj(h_p, w_p, b_p)

        num_r, den_r = _edge_pass(fs_road, fd_road, road_src, road_dst,
                                  pr['attn'])
        num_t, den_t = _edge_pass(fs_tree, fd_tree, tree_src, tree_dst,
                                  pt['attn'])
        num_c, den_c = _edge_pass(fs_cont, fd_cont, cont_src, cont_dst,
                                  pc['attn'])

        h_t = _fin1(num_r, den_r, e_sel)
        scale = 0.5 if li == 1 else 1.0
        h_p = _fin2(num_t, den_t, num_c, den_c, e_sel, scale)
    return h_t, h_p
